# baseline (device time: 63877 ns/iter reference)
import jax
import jax.numpy as jnp
from jax import lax
from jax.experimental import pallas as pl
from jax.experimental.pallas import tpu as pltpu

N_DEV = 32
M = 1024
N = 1024
CH = M // N_DEV
SPLITS = 4
NC = N // SPLITS


def kernel(A, B):
    def body(a_ref, b_ref, out_ref, p_ref, *scratch):
        recv_refs = scratch[0:SPLITS]
        s_sems = scratch[SPLITS:2 * SPLITS]
        r_sems = scratch[2 * SPLITS:3 * SPLITS]
        a_sems = scratch[3 * SPLITS:4 * SPLITS]

        my = lax.axis_index("i")

        barrier = pltpu.get_barrier_semaphore()
        for d in range(1, N_DEV):
            pl.semaphore_signal(
                barrier, inc=1,
                device_id=((my + d) % N_DEV,),
                device_id_type=pl.DeviceIdType.MESH,
            )
        pl.semaphore_wait(barrier, N_DEV - 1)

        p_ref[...] = jnp.dot(
            a_ref[...].astype(jnp.bfloat16),
            b_ref[...].astype(jnp.bfloat16),
            preferred_element_type=jnp.float32,
        ).astype(jnp.bfloat16)

        rs_sends = [[] for _ in range(SPLITS)]
        for h in range(SPLITS):
            for d in range(1, N_DEV):
                t = (my + d) % N_DEV
                rdma = pltpu.make_async_remote_copy(
                    src_ref=p_ref.at[pl.ds(t * CH, CH), pl.ds(h * NC, NC)],
                    dst_ref=recv_refs[h].at[my],
                    send_sem=s_sems[h].at[d - 1],
                    recv_sem=r_sems[h].at[my],
                    device_id=(t,),
                    device_id_type=pl.DeviceIdType.MESH,
                )
                rdma.start()
                rs_sends[h].append(rdma)

        ag_sends = []
        for h in range(SPLITS):
            recv_refs[h][pl.ds(my, 1)] = p_ref[
                pl.ds(my * CH, CH), pl.ds(h * NC, NC)
            ][None]

            for d in range(1, N_DEV):
                s = (my + N_DEV - d) % N_DEV
                rdma = pltpu.make_async_remote_copy(
                    src_ref=recv_refs[h].at[s],
                    dst_ref=recv_refs[h].at[s],
                    send_sem=s_sems[h].at[d - 1],
                    recv_sem=r_sems[h].at[s],
                    device_id=(s,),
                    device_id_type=pl.DeviceIdType.MESH,
                )
                rdma.wait_recv()

            tot = jnp.sum(recv_refs[h][...].astype(jnp.float32), axis=0)
            g = jnp.maximum(tot, 0.0).astype(jnp.bfloat16)
            out_ref[pl.ds(my * CH, CH), pl.ds(h * NC, NC)] = g

            for rdma in rs_sends[h]:
                rdma.wait_send()
            for d in range(1, N_DEV):
                t = (my + d) % N_DEV
                rdma = pltpu.make_async_remote_copy(
                    src_ref=out_ref.at[pl.ds(my * CH, CH), pl.ds(h * NC, NC)],
                    dst_ref=out_ref.at[pl.ds(my * CH, CH), pl.ds(h * NC, NC)],
                    send_sem=s_sems[h].at[d - 1],
                    recv_sem=a_sems[h].at[my],
                    device_id=(t,),
                    device_id_type=pl.DeviceIdType.MESH,
                )
                rdma.start()
                ag_sends.append(rdma)

        for h in range(SPLITS):
            for d in range(1, N_DEV):
                s = (my + N_DEV - d) % N_DEV
                rdma = pltpu.make_async_remote_copy(
                    src_ref=out_ref.at[pl.ds(s * CH, CH), pl.ds(h * NC, NC)],
                    dst_ref=out_ref.at[pl.ds(s * CH, CH), pl.ds(h * NC, NC)],
                    send_sem=s_sems[h].at[d - 1],
                    recv_sem=a_sems[h].at[s],
                    device_id=(s,),
                    device_id_type=pl.DeviceIdType.MESH,
                )
                rdma.wait_recv()

        for rdma in ag_sends:
            rdma.wait_send()

    return pl.pallas_call(
        body,
        out_shape=jax.ShapeDtypeStruct((M, N), jnp.bfloat16),
        in_specs=[
            pl.BlockSpec(memory_space=pltpu.VMEM),
            pl.BlockSpec(memory_space=pltpu.VMEM),
        ],
        out_specs=pl.BlockSpec(memory_space=pltpu.VMEM),
        scratch_shapes=(
            [pltpu.VMEM((M, N), jnp.bfloat16)]
            + [pltpu.VMEM((N_DEV, CH, NC), jnp.bfloat16)] * SPLITS
            + [pltpu.SemaphoreType.DMA((N_DEV - 1,))] * SPLITS
            + [pltpu.SemaphoreType.DMA((N_DEV,))] * SPLITS
            + [pltpu.SemaphoreType.DMA((N_DEV,))] * SPLITS
        ),
        compiler_params=pltpu.CompilerParams(collective_id=0),
    )(A, B)


# device time: 62140 ns/iter; 1.0280x vs baseline; 1.0280x over previous
import jax
import jax.numpy as jnp
from jax import lax
from jax.experimental import pallas as pl
from jax.experimental.pallas import tpu as pltpu

N_DEV = 32
M = 1024
N = 1024
CH = M // N_DEV
SPLITS = 2
NC = N // SPLITS


def kernel(A, B):
    def body(a_ref, b_ref, out_ref, p_ref, *scratch):
        recv_refs = scratch[0:SPLITS]
        s_sems = scratch[SPLITS:2 * SPLITS]
        r_sems = scratch[2 * SPLITS:3 * SPLITS]
        a_sems = scratch[3 * SPLITS:4 * SPLITS]

        my = lax.axis_index("i")

        barrier = pltpu.get_barrier_semaphore()
        for d in range(1, N_DEV):
            pl.semaphore_signal(
                barrier, inc=1,
                device_id=((my + d) % N_DEV,),
                device_id_type=pl.DeviceIdType.MESH,
            )

        a_bf = a_ref[...].astype(jnp.bfloat16)

        rs_sends = [[] for _ in range(SPLITS)]
        for h in range(SPLITS):
            p_ref[:, pl.ds(h * NC, NC)] = jnp.dot(
                a_bf,
                b_ref[:, pl.ds(h * NC, NC)].astype(jnp.bfloat16),
                preferred_element_type=jnp.float32,
            ).astype(jnp.bfloat16)
            if h == 0:
                pl.semaphore_wait(barrier, N_DEV - 1)
            for d in range(1, N_DEV):
                t = (my + d) % N_DEV
                rdma = pltpu.make_async_remote_copy(
                    src_ref=p_ref.at[pl.ds(t * CH, CH), pl.ds(h * NC, NC)],
                    dst_ref=recv_refs[h].at[my],
                    send_sem=s_sems[h].at[d - 1],
                    recv_sem=r_sems[h].at[my],
                    device_id=(t,),
                    device_id_type=pl.DeviceIdType.MESH,
                )
                rdma.start()
                rs_sends[h].append(rdma)

        ag_sends = []
        for h in range(SPLITS):
            recv_refs[h][pl.ds(my, 1)] = p_ref[
                pl.ds(my * CH, CH), pl.ds(h * NC, NC)
            ][None]

            for d in range(1, N_DEV):
                s = (my + N_DEV - d) % N_DEV
                rdma = pltpu.make_async_remote_copy(
                    src_ref=recv_refs[h].at[s],
                    dst_ref=recv_refs[h].at[s],
                    send_sem=s_sems[h].at[d - 1],
                    recv_sem=r_sems[h].at[s],
                    device_id=(s,),
                    device_id_type=pl.DeviceIdType.MESH,
                )
                rdma.wait_recv()

            tot = jnp.sum(recv_refs[h][...].astype(jnp.float32), axis=0)
            g = jnp.maximum(tot, 0.0).astype(jnp.bfloat16)
            out_ref[pl.ds(my * CH, CH), pl.ds(h * NC, NC)] = g

            for rdma in rs_sends[h]:
                rdma.wait_send()
            for d in range(1, N_DEV):
                t = (my + d) % N_DEV
                rdma = pltpu.make_async_remote_copy(
                    src_ref=out_ref.at[pl.ds(my * CH, CH), pl.ds(h * NC, NC)],
                    dst_ref=out_ref.at[pl.ds(my * CH, CH), pl.ds(h * NC, NC)],
                    send_sem=s_sems[h].at[d - 1],
                    recv_sem=a_sems[h].at[my],
                    device_id=(t,),
                    device_id_type=pl.DeviceIdType.MESH,
                )
                rdma.start()
                ag_sends.append(rdma)

        for h in range(SPLITS):
            for d in range(1, N_DEV):
                s = (my + N_DEV - d) % N_DEV
                rdma = pltpu.make_async_remote_copy(
                    src_ref=out_ref.at[pl.ds(s * CH, CH), pl.ds(h * NC, NC)],
                    dst_ref=out_ref.at[pl.ds(s * CH, CH), pl.ds(h * NC, NC)],
                    send_sem=s_sems[h].at[d - 1],
                    recv_sem=a_sems[h].at[s],
                    device_id=(s,),
                    device_id_type=pl.DeviceIdType.MESH,
                )
                rdma.wait_recv()

        for rdma in ag_sends:
            rdma.wait_send()

    return pl.pallas_call(
        body,
        out_shape=jax.ShapeDtypeStruct((M, N), jnp.bfloat16),
        in_specs=[
            pl.BlockSpec(memory_space=pltpu.VMEM),
            pl.BlockSpec(memory_space=pltpu.VMEM),
        ],
        out_specs=pl.BlockSpec(memory_space=pltpu.VMEM),
        scratch_shapes=(
            [pltpu.VMEM((M, N), jnp.bfloat16)]
            + [pltpu.VMEM((N_DEV, CH, NC), jnp.bfloat16)] * SPLITS
            + [pltpu.SemaphoreType.DMA((N_DEV - 1,))] * SPLITS
            + [pltpu.SemaphoreType.DMA((N_DEV,))] * SPLITS
            + [pltpu.SemaphoreType.DMA((N_DEV,))] * SPLITS
        ),
        compiler_params=pltpu.CompilerParams(collective_id=0),
    )(A, B)
